# baseline (device time: 23402 ns/iter reference)
import jax
import jax.numpy as jnp
from jax import lax
from jax.experimental import pallas as pl
from jax.experimental.pallas import tpu as pltpu

N_DEV = 32
N_CHUNK = 4
FP8 = jnp.float8_e4m3fn


def kernel(x, w_mat, scale_x, scale_w):
    m_total, k_shard = x.shape
    k_total, n = w_mat.shape
    blk = m_total // N_DEV
    kc = k_total // N_CHUNK
    bc = N_DEV // N_CHUNK

    def body(x_ref, w_hbm_ref, sx_ref, sw_ref, out_ref,
             x8_ref, xrow_ref, w32_ref, w8_ref,
             send_sems, recv_sems, w_sems):
        my = lax.axis_index("i")

        w_dmas = []
        for q in range(N_CHUNK):
            d = pltpu.make_async_copy(
                w_hbm_ref.at[pl.ds(q * kc, kc), :],
                w32_ref.at[pl.ds(q * kc, kc), :],
                w_sems.at[q],
            )
            d.start()
            w_dmas.append(d)

        x8_ref[...] = x_ref[...].astype(FP8)

        bsem = pltpu.get_barrier_semaphore()
        for i in range(N_DEV):
            @pl.when(i != my)
            def _():
                pl.semaphore_signal(
                    bsem, inc=1,
                    device_id=(i,),
                    device_id_type=pl.DeviceIdType.MESH,
                )
        pl.semaphore_wait(bsem, N_DEV - 1)

        pltpu.make_async_copy(
            x8_ref.at[pl.ds(my * blk, blk), :],
            xrow_ref.at[:, pl.ds(my * blk, blk)],
            recv_sems.at[my],
        ).start()

        for k in range(1, N_DEV):
            i = lax.rem(my + k, N_DEV)
            pltpu.make_async_remote_copy(
                src_ref=x8_ref.at[pl.ds(i * blk, blk), :],
                dst_ref=xrow_ref.at[:, pl.ds(my * blk, blk)],
                send_sem=send_sems.at[k],
                recv_sem=recv_sems.at[my],
                device_id=(i,),
                device_id_type=pl.DeviceIdType.MESH,
            ).start()

        sxw = sx_ref[0] * sw_ref[0]
        for q in range(N_CHUNK):
            w_dmas[q].wait()
            w8_ref[pl.ds(q * kc, kc), :] = (
                w32_ref[pl.ds(q * kc, kc), :].astype(FP8))
            for j in range(q * bc, (q + 1) * bc):
                pltpu.make_async_copy(
                    xrow_ref.at[:, pl.ds(j * blk, blk)],
                    xrow_ref.at[:, pl.ds(j * blk, blk)],
                    recv_sems.at[j],
                ).wait()
            part = jnp.dot(
                xrow_ref[:, pl.ds(q * kc, kc)],
                w8_ref[pl.ds(q * kc, kc), :],
                preferred_element_type=jnp.float32,
            )
            if q == 0:
                out_ref[...] = part
            elif q < N_CHUNK - 1:
                out_ref[...] = out_ref[...] + part
            else:
                out_ref[...] = (out_ref[...] + part) * sxw

        for k in range(1, N_DEV):
            pltpu.make_async_copy(
                x8_ref.at[pl.ds(0, blk), :],
                x8_ref.at[pl.ds(0, blk), :],
                send_sems.at[k],
            ).wait()

    return pl.pallas_call(
        body,
        out_shape=jax.ShapeDtypeStruct((blk, n), jnp.float32),
        in_specs=[
            pl.BlockSpec(memory_space=pltpu.VMEM),
            pl.BlockSpec(memory_space=pl.ANY),
            pl.BlockSpec(memory_space=pltpu.SMEM),
            pl.BlockSpec(memory_space=pltpu.SMEM),
        ],
        out_specs=pl.BlockSpec(memory_space=pltpu.VMEM),
        scratch_shapes=[
            pltpu.VMEM((m_total, k_shard), FP8),
            pltpu.VMEM((blk, k_total), FP8),
            pltpu.VMEM((k_total, n), jnp.float32),
            pltpu.VMEM((k_total, n), FP8),
            pltpu.SemaphoreType.DMA((N_DEV,)),
            pltpu.SemaphoreType.DMA((N_DEV,)),
            pltpu.SemaphoreType.DMA((N_CHUNK,)),
        ],
        compiler_params=pltpu.CompilerParams(
            collective_id=0,
            vmem_limit_bytes=100 * 1024 * 1024,
        ),
    )(x, w_mat, scale_x, scale_w)


# device time: 22067 ns/iter; 1.0605x vs baseline; 1.0605x over previous
import jax
import jax.numpy as jnp
from jax import lax
from jax.experimental import pallas as pl
from jax.experimental.pallas import tpu as pltpu

N_DEV = 32
FP8 = jnp.float8_e4m3fn


def kernel(x, w_mat, scale_x, scale_w):
    m_total, k_shard = x.shape
    k_total, n = w_mat.shape
    blk = m_total // N_DEV

    def body(x_ref, w_hbm_ref, sx_ref, sw_ref, out_ref,
             x8_ref, xrow_ref, w32_ref, w8_ref,
             send_sems, recv_sems, w_sems):
        my = lax.axis_index("i")

        w_dma = pltpu.make_async_copy(w_hbm_ref, w32_ref, w_sems.at[0])
        w_dma.start()

        x8_ref[...] = x_ref[...].astype(FP8)

        bsem = pltpu.get_barrier_semaphore()
        for i in range(N_DEV):
            @pl.when(i != my)
            def _():
                pl.semaphore_signal(
                    bsem, inc=1,
                    device_id=(i,),
                    device_id_type=pl.DeviceIdType.MESH,
                )
        pl.semaphore_wait(bsem, N_DEV - 1)

        pltpu.make_async_copy(
            x8_ref.at[pl.ds(my * blk, blk), :],
            xrow_ref.at[:, pl.ds(my * blk, blk)],
            recv_sems.at[my],
        ).start()

        for k in range(1, N_DEV):
            i = lax.rem(my + k, N_DEV)
            pltpu.make_async_remote_copy(
                src_ref=x8_ref.at[pl.ds(i * blk, blk), :],
                dst_ref=xrow_ref.at[:, pl.ds(my * blk, blk)],
                send_sem=send_sems.at[k],
                recv_sem=recv_sems.at[my],
                device_id=(i,),
                device_id_type=pl.DeviceIdType.MESH,
            ).start()

        w_dma.wait()
        w8_ref[...] = w32_ref[...].astype(FP8)

        half = N_DEV // 2
        for j in range(half):
            pltpu.make_async_copy(
                xrow_ref.at[:, pl.ds(j * blk, blk)],
                xrow_ref.at[:, pl.ds(j * blk, blk)],
                recv_sems.at[j],
            ).wait()
        acc = jnp.dot(xrow_ref[:, : half * blk], w8_ref[: half * blk, :],
                      preferred_element_type=jnp.float32)

        for j in range(half, N_DEV):
            pltpu.make_async_copy(
                xrow_ref.at[:, pl.ds(j * blk, blk)],
                xrow_ref.at[:, pl.ds(j * blk, blk)],
                recv_sems.at[j],
            ).wait()
        acc = acc + jnp.dot(xrow_ref[:, half * blk :], w8_ref[half * blk :, :],
                            preferred_element_type=jnp.float32)
        out_ref[...] = acc * (sx_ref[0] * sw_ref[0])

        for k in range(1, N_DEV):
            pltpu.make_async_copy(
                x8_ref.at[pl.ds(0, blk), :],
                x8_ref.at[pl.ds(0, blk), :],
                send_sems.at[k],
            ).wait()

    return pl.pallas_call(
        body,
        out_shape=jax.ShapeDtypeStruct((blk, n), jnp.float32),
        in_specs=[
            pl.BlockSpec(memory_space=pltpu.VMEM),
            pl.BlockSpec(memory_space=pl.ANY),
            pl.BlockSpec(memory_space=pltpu.SMEM),
            pl.BlockSpec(memory_space=pltpu.SMEM),
        ],
        out_specs=pl.BlockSpec(memory_space=pltpu.VMEM),
        scratch_shapes=[
            pltpu.VMEM((m_total, k_shard), FP8),
            pltpu.VMEM((blk, k_total), FP8),
            pltpu.VMEM((k_total, n), jnp.float32),
            pltpu.VMEM((k_total, n), FP8),
            pltpu.SemaphoreType.DMA((N_DEV,)),
            pltpu.SemaphoreType.DMA((N_DEV,)),
            pltpu.SemaphoreType.DMA((1,)),
        ],
        compiler_params=pltpu.CompilerParams(
            collective_id=0,
            vmem_limit_bytes=100 * 1024 * 1024,
        ),
    )(x, w_mat, scale_x, scale_w)
